# baseline (speedup 1.0000x reference)
import functools

import jax
import jax.numpy as jnp
from jax import lax
from jax.experimental import pallas as pl
from jax.experimental.pallas import tpu as pltpu
from jax.experimental.pallas import tpu_sc as plsc

_B = 16384
_D = 64
_OUT = 128
_V = 1000000

_info = plsc.get_sparse_core_info()
_NC, _NS = _info.num_cores, _info.num_subcores
_NW = _NC * _NS
_BPW = _B // _NW


_CV = 40960
_NSTEP = -(-_V // _CV)
_Q = _CV // 4
_VP = _NSTEP * _Q


def _reformat_body(bt_ref, o_ref):
    t = jnp.transpose(bt_ref[...])
    u = lax.bitcast_convert_type(
        t.astype(jnp.bfloat16), jnp.uint16
    ).astype(jnp.int32)
    o_ref[:, :_D] = jnp.bitwise_or(u[0:_Q], u[_Q : 2 * _Q] << 16)
    o_ref[:, _D:] = jnp.bitwise_or(u[2 * _Q : 3 * _Q], u[3 * _Q :] << 16)


def _tc_reformat(bankT):
    return pl.pallas_call(
        _reformat_body,
        grid=(_NSTEP,),
        in_specs=[pl.BlockSpec((_D, _CV), lambda i: (0, i))],
        out_specs=pl.BlockSpec((_Q, 2 * _D), lambda i: (i, 0)),
        out_shape=jax.ShapeDtypeStruct((_VP, 2 * _D), jnp.int32),
    )(bankT)


@functools.partial(
    pl.kernel,
    mesh=plsc.VectorSubcoreMesh(core_axis_name="c", subcore_axis_name="s"),
    out_type=jax.ShapeDtypeStruct((_B, 2 * _D), jnp.int32),
    scratch_types=[
        pltpu.VMEM((_BPW,), jnp.int32),
        pltpu.VMEM((_BPW, 2 * _D), jnp.int32),
        pltpu.SemaphoreType.DMA,
    ],
)
def _sc_gather(bank_hbm, idx_hbm, out_hbm, idx_v, rows_v, sem):
    wid = lax.axis_index("s") * _NC + lax.axis_index("c")
    base = wid * _BPW
    pltpu.sync_copy(idx_hbm.at[pl.ds(base, _BPW)], idx_v)
    pltpu.async_copy(bank_hbm.at[idx_v], rows_v, sem).wait()
    pltpu.sync_copy(rows_v, out_hbm.at[pl.ds(base, _BPW)])


def _mm_body(emb_ref, qi_ref, w_ref, b_ref, o_ref):
    qi = qi_ref[...]
    g = jnp.where(qi >= 2, emb_ref[:, _D:], emb_ref[:, :_D])
    packed = jnp.where((qi & 1) == 1, g & jnp.int32(-65536), g << 16)
    sel = lax.bitcast_convert_type(packed, jnp.float32)
    o_ref[...] = (
        jnp.dot(sel, w_ref[...], preferred_element_type=jnp.float32)
        + b_ref[...]
    )


_BM = 2048


def _tc_project(emb2, qi2d, W, b2d):
    return pl.pallas_call(
        _mm_body,
        grid=(_B // _BM,),
        in_specs=[
            pl.BlockSpec((_BM, 2 * _D), lambda i: (i, 0)),
            pl.BlockSpec((_BM, 1), lambda i: (i, 0)),
            pl.BlockSpec((_D, _OUT), lambda i: (0, 0)),
            pl.BlockSpec((1, _OUT), lambda i: (0, 0)),
        ],
        out_specs=pl.BlockSpec((_BM, _OUT), lambda i: (i, 0)),
        out_shape=jax.ShapeDtypeStruct((_B, _OUT), jnp.float32),
    )(emb2, qi2d, W, b2d)


def kernel(x, bank, W, b):
    idx = jnp.squeeze(x).astype(jnp.int32)
    c = idx // _CV
    r = idx % _CV
    phys = c * _Q + (r % _Q)
    qi2d = (r // _Q).reshape(_B, 1)
    bankP = _tc_reformat(bank.T)
    emb2 = _sc_gather(bankP, phys)
    return _tc_project(emb2, qi2d, W, b.reshape(1, _OUT))


# speedup vs baseline: 1.0233x; 1.0233x over previous
import functools

import jax
import jax.numpy as jnp
from jax import lax
from jax.experimental import pallas as pl
from jax.experimental.pallas import tpu as pltpu
from jax.experimental.pallas import tpu_sc as plsc

_B = 16384
_D = 64
_OUT = 128
_V = 1000000

_info = plsc.get_sparse_core_info()
_NC, _NS = _info.num_cores, _info.num_subcores
_NW = _NC * _NS
_BPW = _B // _NW


_CV = 49152
_NSTEP = -(-_V // _CV)
_Q = _CV // 4
_VP = _NSTEP * _Q


def _reformat_body(bt_ref, o_ref):
    t = jnp.transpose(bt_ref[...])
    u = lax.bitcast_convert_type(
        t.astype(jnp.bfloat16), jnp.uint16
    ).astype(jnp.int32)
    o_ref[:, :_D] = jnp.bitwise_or(u[0:_Q], u[_Q : 2 * _Q] << 16)
    o_ref[:, _D:] = jnp.bitwise_or(u[2 * _Q : 3 * _Q], u[3 * _Q :] << 16)


def _tc_reformat(bankT):
    return pl.pallas_call(
        _reformat_body,
        grid=(_NSTEP,),
        in_specs=[pl.BlockSpec((_D, _CV), lambda i: (0, i))],
        out_specs=pl.BlockSpec((_Q, 2 * _D), lambda i: (i, 0)),
        out_shape=jax.ShapeDtypeStruct((_VP, 2 * _D), jnp.int32),
    )(bankT)


@functools.partial(
    pl.kernel,
    mesh=plsc.VectorSubcoreMesh(core_axis_name="c", subcore_axis_name="s"),
    out_type=jax.ShapeDtypeStruct((_B, 2 * _D), jnp.int32),
    scratch_types=[
        pltpu.VMEM((_BPW,), jnp.int32),
        pltpu.VMEM((_BPW, 2 * _D), jnp.int32),
        pltpu.SemaphoreType.DMA,
    ],
)
def _sc_gather(bank_hbm, idx_hbm, out_hbm, idx_v, rows_v, sem):
    wid = lax.axis_index("s") * _NC + lax.axis_index("c")
    base = wid * _BPW
    pltpu.sync_copy(idx_hbm.at[pl.ds(base, _BPW)], idx_v)
    pltpu.async_copy(bank_hbm.at[idx_v], rows_v, sem).wait()
    pltpu.sync_copy(rows_v, out_hbm.at[pl.ds(base, _BPW)])


def _mm_body(emb_ref, qi_ref, w_ref, b_ref, o_ref):
    qi = qi_ref[...]
    g = jnp.where(qi >= 2, emb_ref[:, _D:], emb_ref[:, :_D])
    packed = jnp.where((qi & 1) == 1, g & jnp.int32(-65536), g << 16)
    sel = lax.bitcast_convert_type(packed, jnp.float32)
    o_ref[...] = (
        jnp.dot(sel, w_ref[...], preferred_element_type=jnp.float32)
        + b_ref[...]
    )


_BM = 4096


def _tc_project(emb2, qi2d, W, b2d):
    return pl.pallas_call(
        _mm_body,
        grid=(_B // _BM,),
        in_specs=[
            pl.BlockSpec((_BM, 2 * _D), lambda i: (i, 0)),
            pl.BlockSpec((_BM, 1), lambda i: (i, 0)),
            pl.BlockSpec((_D, _OUT), lambda i: (0, 0)),
            pl.BlockSpec((1, _OUT), lambda i: (0, 0)),
        ],
        out_specs=pl.BlockSpec((_BM, _OUT), lambda i: (i, 0)),
        out_shape=jax.ShapeDtypeStruct((_B, _OUT), jnp.float32),
    )(emb2, qi2d, W, b2d)


def kernel(x, bank, W, b):
    idx = jnp.squeeze(x).astype(jnp.int32)
    c = idx // _CV
    r = idx % _CV
    phys = c * _Q + (r % _Q)
    qi2d = (r // _Q).reshape(_B, 1)
    bankP = _tc_reformat(bank.T)
    emb2 = _sc_gather(bankP, phys)
    return _tc_project(emb2, qi2d, W, b.reshape(1, _OUT))
